# Initial kernel scaffold; baseline (speedup 1.0000x reference)
import jax
import jax.numpy as jnp
from jax.experimental import pallas as pl


def kernel(grid_vertices, grid_features, point_vertices, point_features, eW1, eb1, eg, ebeta, eW2, eb2, oW1, ob1, og, obeta, oW2, ob2):
    raise NotImplementedError("write your pallas kernel here")


# speedup vs baseline: 6.3908x; 6.3908x over previous
import functools

import jax
import jax.numpy as jnp
from jax.experimental import pallas as pl
from jax.experimental.pallas import tpu as pltpu
from jax.experimental.pallas import tpu_sc as plsc

NG = 32768
NPTS = 10000
NPP = 10240
KNN = 16
BQ = 256
NBLK = NPP // BQ
RCH = 512
NCH = NG // RCH
TOPC = 4
NCAND = NCH * TOPC
TW = 144
SCALE = 32.0


def _knn_body(qt_ref, rp_ref, out_ref, cv_ref, ci_ref):
    big_i = jnp.int32(2 ** 30)
    inf_f = jnp.float32(jnp.inf)

    def chunk(c):
        d = jnp.zeros((RCH, BQ), jnp.float32)
        for cc in range(3):
            rc = rp_ref[pl.ds(c * RCH, RCH), cc:cc + 1]
            qc = qt_ref[cc:cc + 1, :]
            t = rc - qc
            d = d + t * t
        riota = (jax.lax.broadcasted_iota(jnp.int32, (RCH, BQ), 0)
                 + c * RCH)
        for j in range(TOPC):
            m = jnp.min(d, axis=0, keepdims=True)
            am = jnp.min(jnp.where(d <= m, riota, big_i),
                         axis=0, keepdims=True)
            cv_ref[pl.ds(c * TOPC + j, 1), :] = m
            ci_ref[pl.ds(c * TOPC + j, 1), :] = am
            if j + 1 < TOPC:
                d = jnp.where(riota == am, inf_f, d)

    for c in range(NCH):
        chunk(c)

    cv = cv_ref[...]
    ci = ci_ref[...]
    sio = jax.lax.broadcasted_iota(jnp.int32, (NCAND, BQ), 0)
    for j in range(KNN):
        m = jnp.min(cv, axis=0, keepdims=True)
        aslot = jnp.min(jnp.where(cv <= m, sio, big_i),
                        axis=0, keepdims=True)
        sel = sio == aslot
        gidx = jnp.min(jnp.where(sel, ci, big_i), axis=0, keepdims=True)
        out_ref[pl.ds(j, 1), :] = gidx
        if j + 1 < KNN:
            cv = jnp.where(sel, inf_f, cv)


def _knn(qt, rp):
    return pl.pallas_call(
        _knn_body,
        grid=(NBLK,),
        in_specs=[
            pl.BlockSpec((8, BQ), lambda i: (0, i)),
            pl.BlockSpec((NG, 8), lambda i: (0, 0)),
        ],
        out_specs=pl.BlockSpec((KNN, BQ), lambda i: (0, i)),
        out_shape=jax.ShapeDtypeStruct((KNN, NPP), jnp.int32),
        scratch_shapes=[
            pltpu.VMEM((NCAND, BQ), jnp.float32),
            pltpu.VMEM((NCAND, BQ), jnp.int32),
        ],
    )(qt, rp)


def _sc_gather(table, idx_flat):
    e = idx_flat.shape[0]
    info = plsc.get_sparse_core_info()
    nw = info.num_cores * info.num_subcores
    per_w = e // nw
    ch = 128
    n_iter = per_w // ch
    mesh = plsc.VectorSubcoreMesh(core_axis_name="c", subcore_axis_name="s")

    @functools.partial(
        pl.kernel, mesh=mesh,
        compiler_params=pltpu.CompilerParams(use_tc_tiling_on_sc=False),
        out_type=jax.ShapeDtypeStruct((e, TW), jnp.float32),
        scratch_types=[
            pltpu.VMEM((ch,), jnp.int32),
            pltpu.VMEM((ch, TW), jnp.float32),
            pltpu.SemaphoreType.DMA,
        ],
    )
    def k(table_hbm, idx_hbm, out_hbm, idx_v, rows_v, sem):
        wid = (jax.lax.axis_index("s") * info.num_cores
               + jax.lax.axis_index("c"))
        base = wid * per_w

        def body(i, _):
            off = base + i * ch
            pltpu.sync_copy(idx_hbm.at[pl.ds(off, ch)], idx_v)
            pltpu.async_copy(table_hbm.at[idx_v], rows_v, sem).wait()
            pltpu.sync_copy(rows_v, out_hbm.at[pl.ds(off, ch)])
            return 0

        jax.lax.fori_loop(0, n_iter, body, 0)

    return k(table, idx_flat)


def _ln_gelu(h, g, beta):
    mu = jnp.mean(h, axis=-1, keepdims=True)
    var = jnp.mean((h - mu) ** 2, axis=-1, keepdims=True)
    h = (h - mu) / jnp.sqrt(var + 1e-5) * g + beta
    return jax.nn.gelu(h)


def _mlp_body(g3_ref, pf_ref, qv_ref,
              w1cat_ref, w1self_ref, w1r_ref, eb1_ref, eg_ref, ebeta_ref,
              ew2_ref, eb2_ref, ow1_ref, ob1_ref, og_ref, obeta_ref,
              ow2_ref, ob2_ref, out_ref):
    mm = functools.partial(jnp.dot, preferred_element_type=jnp.float32)
    pf = pf_ref[...]
    qv = qv_ref[...]
    hself = (mm(pf, w1self_ref[...]) - mm(qv, w1r_ref[...])
             + eb1_ref[...])
    w1cat = w1cat_ref[...]
    ew2 = ew2_ref[...]
    eg = eg_ref[...]
    ebeta = ebeta_ref[...]
    acc = jnp.zeros((BQ, 128), jnp.float32)
    for k in range(KNN):
        gk = g3_ref[k]
        h = mm(gk, w1cat) + hself
        h = _ln_gelu(h, eg, ebeta)
        acc = acc + mm(h, ew2)
    red = acc * (1.0 / KNN) + eb2_ref[...]
    h2 = mm(red, ow1_ref[...]) + ob1_ref[...]
    h2 = _ln_gelu(h2, og_ref[...], obeta_ref[...])
    out_ref[...] = mm(h2, ow2_ref[...]) + ob2_ref[...]


def _mlp(g3, pf, qv16, w1cat, w1self, w1r16, eb1, eg, ebeta, ew2, eb2,
         ow1, ob1, og, obeta, ow2, ob2):
    full = lambda shape: pl.BlockSpec(shape, lambda i: tuple(0 for _ in shape))
    return pl.pallas_call(
        _mlp_body,
        grid=(NBLK,),
        in_specs=[
            pl.BlockSpec((KNN, BQ, TW), lambda i: (0, i, 0)),
            pl.BlockSpec((BQ, 128), lambda i: (i, 0)),
            pl.BlockSpec((BQ, 16), lambda i: (i, 0)),
            full((TW, 256)), full((128, 256)), full((16, 256)),
            full((1, 256)), full((1, 256)), full((1, 256)),
            full((256, 128)), full((1, 128)),
            full((128, 256)), full((1, 256)), full((1, 256)), full((1, 256)),
            full((256, 128)), full((1, 128)),
        ],
        out_specs=pl.BlockSpec((BQ, 128), lambda i: (i, 0)),
        out_shape=jax.ShapeDtypeStruct((NPP, 128), jnp.float32),
    )(g3, pf, qv16, w1cat, w1self, w1r16, eb1, eg, ebeta, ew2, eb2,
      ow1, ob1, og, obeta, ow2, ob2)


def kernel(grid_vertices, grid_features, point_vertices, point_features,
           eW1, eb1, eg, ebeta, eW2, eb2,
           oW1, ob1, og, obeta, oW2, ob2):
    f32 = jnp.float32
    qv = point_vertices * SCALE
    inv = grid_vertices * SCALE

    qt = jnp.zeros((8, NPP), f32).at[:3, :NPTS].set(qv.T)
    rp = jnp.zeros((NG, 8), f32).at[:, :3].set(inv)
    idx = _knn(qt, rp)

    table = (jnp.zeros((NG, TW), f32)
             .at[:, :128].set(grid_features)
             .at[:, 128:131].set(inv))
    g = _sc_gather(table, idx.reshape(-1))
    g3 = g.reshape(KNN, NPP, TW)

    pf_pad = jnp.pad(point_features, ((0, NPP - NPTS), (0, 0)))
    qv16 = jnp.zeros((NPP, 16), f32).at[:NPTS, :3].set(qv)
    w1cat = (jnp.zeros((TW, 256), f32)
             .at[:128].set(eW1[:128])
             .at[128:131].set(eW1[256:259]))
    w1self = eW1[128:256]
    w1r16 = jnp.zeros((16, 256), f32).at[:3].set(eW1[256:259])

    out = _mlp(g3, pf_pad, qv16, w1cat, w1self, w1r16,
               eb1[None], eg[None], ebeta[None], eW2, eb2[None],
               oW1, ob1[None], og[None], obeta[None], oW2, ob2[None])
    return out[:NPTS]


# speedup vs baseline: 10.5541x; 1.6514x over previous
import functools

import jax
import jax.numpy as jnp
from jax.experimental import pallas as pl
from jax.experimental.pallas import tpu as pltpu
from jax.experimental.pallas import tpu_sc as plsc

NG = 32768
NPTS = 10000
NPP = 10240
KNN = 16
BQ = 256
NBLK = NPP // BQ
RCH = 256
NCH = NG // RCH
TOPC = 3
NCAND = NCH * TOPC
TW = 144
SCALE = 32.0


def _knn_body(qt_ref, rp_ref, out_ref, ck_ref):
    big_i = jnp.int32(2 ** 31 - 1)
    lidx = jax.lax.broadcasted_iota(jnp.int32, (RCH, BQ), 0)

    def chunk(c):
        d = None
        for cc in range(3):
            rc = rp_ref[pl.ds(c * RCH, RCH), cc:cc + 1]
            qc = qt_ref[cc:cc + 1, :]
            t = rc - qc
            d = t * t if d is None else d + t * t
        db = jax.lax.bitcast_convert_type(d, jnp.int32)
        key = jnp.bitwise_or(jnp.bitwise_and(db, jnp.int32(-256)), lidx)
        for j in range(TOPC):
            m = jnp.min(key, axis=0, keepdims=True)
            ck_ref[pl.ds(c * TOPC + j, 1), :] = m
            if j + 1 < TOPC:
                key = jnp.where(key == m, big_i, key)

    for c in range(NCH):
        chunk(c)

    ck = ck_ref[...]
    sio = jax.lax.broadcasted_iota(jnp.int32, (NCAND, BQ), 0)
    for j in range(KNN):
        m = jnp.min(ck, axis=0, keepdims=True)
        aslot = jnp.min(jnp.where(ck == m, sio, big_i),
                        axis=0, keepdims=True)
        gidx = (aslot // TOPC) * RCH + jnp.bitwise_and(m, jnp.int32(255))
        out_ref[pl.ds(j, 1), :] = gidx
        if j + 1 < KNN:
            ck = jnp.where(sio == aslot, big_i, ck)


def _knn(qt, rp):
    return pl.pallas_call(
        _knn_body,
        grid=(NBLK,),
        in_specs=[
            pl.BlockSpec((8, BQ), lambda i: (0, i)),
            pl.BlockSpec((NG, 8), lambda i: (0, 0)),
        ],
        out_specs=pl.BlockSpec((KNN, BQ), lambda i: (0, i)),
        out_shape=jax.ShapeDtypeStruct((KNN, NPP), jnp.int32),
        scratch_shapes=[
            pltpu.VMEM((NCAND, BQ), jnp.int32),
        ],
    )(qt, rp)


def _sc_gather(table, idx_flat):
    e = idx_flat.shape[0]
    info = plsc.get_sparse_core_info()
    nw = info.num_cores * info.num_subcores
    per_w = e // nw
    ch = 128
    n_iter = per_w // ch
    mesh = plsc.VectorSubcoreMesh(core_axis_name="c", subcore_axis_name="s")

    @functools.partial(
        pl.kernel, mesh=mesh,
        compiler_params=pltpu.CompilerParams(use_tc_tiling_on_sc=False),
        out_type=jax.ShapeDtypeStruct((e, TW), jnp.float32),
        scratch_types=[
            pltpu.VMEM((ch,), jnp.int32),
            pltpu.VMEM((ch, TW), jnp.float32),
            pltpu.SemaphoreType.DMA,
        ],
    )
    def k(table_hbm, idx_hbm, out_hbm, idx_v, rows_v, sem):
        wid = (jax.lax.axis_index("s") * info.num_cores
               + jax.lax.axis_index("c"))
        base = wid * per_w

        def body(i, _):
            off = base + i * ch
            pltpu.sync_copy(idx_hbm.at[pl.ds(off, ch)], idx_v)
            pltpu.async_copy(table_hbm.at[idx_v], rows_v, sem).wait()
            pltpu.sync_copy(rows_v, out_hbm.at[pl.ds(off, ch)])
            return 0

        jax.lax.fori_loop(0, n_iter, body, 0)

    return k(table, idx_flat)


def _ln_gelu(h, g, beta):
    mu = jnp.mean(h, axis=-1, keepdims=True)
    var = jnp.mean((h - mu) ** 2, axis=-1, keepdims=True)
    h = (h - mu) / jnp.sqrt(var + 1e-5) * g + beta
    return jax.nn.gelu(h)


def _mlp_body(g3_ref, pf_ref, qv_ref,
              w1cat_ref, w1self_ref, w1r_ref, eb1_ref, eg_ref, ebeta_ref,
              ew2_ref, eb2_ref, ow1_ref, ob1_ref, og_ref, obeta_ref,
              ow2_ref, ob2_ref, out_ref):
    mm = functools.partial(jnp.dot, preferred_element_type=jnp.float32)
    pf = pf_ref[...]
    qv = qv_ref[...]
    hself = (mm(pf, w1self_ref[...]) - mm(qv, w1r_ref[...])
             + eb1_ref[...])
    w1cat = w1cat_ref[...]
    ew2 = ew2_ref[...]
    eg = eg_ref[...]
    ebeta = ebeta_ref[...]
    acc = jnp.zeros((BQ, 128), jnp.float32)
    for k in range(KNN):
        gk = g3_ref[k]
        h = mm(gk, w1cat) + hself
        h = _ln_gelu(h, eg, ebeta)
        acc = acc + mm(h, ew2)
    red = acc * (1.0 / KNN) + eb2_ref[...]
    h2 = mm(red, ow1_ref[...]) + ob1_ref[...]
    h2 = _ln_gelu(h2, og_ref[...], obeta_ref[...])
    out_ref[...] = mm(h2, ow2_ref[...]) + ob2_ref[...]


def _mlp(g3, pf, qv16, w1cat, w1self, w1r16, eb1, eg, ebeta, ew2, eb2,
         ow1, ob1, og, obeta, ow2, ob2):
    full = lambda shape: pl.BlockSpec(shape, lambda i: tuple(0 for _ in shape))
    return pl.pallas_call(
        _mlp_body,
        grid=(NBLK,),
        in_specs=[
            pl.BlockSpec((KNN, BQ, TW), lambda i: (0, i, 0)),
            pl.BlockSpec((BQ, 128), lambda i: (i, 0)),
            pl.BlockSpec((BQ, 16), lambda i: (i, 0)),
            full((TW, 256)), full((128, 256)), full((16, 256)),
            full((1, 256)), full((1, 256)), full((1, 256)),
            full((256, 128)), full((1, 128)),
            full((128, 256)), full((1, 256)), full((1, 256)), full((1, 256)),
            full((256, 128)), full((1, 128)),
        ],
        out_specs=pl.BlockSpec((BQ, 128), lambda i: (i, 0)),
        out_shape=jax.ShapeDtypeStruct((NPP, 128), jnp.float32),
    )(g3, pf, qv16, w1cat, w1self, w1r16, eb1, eg, ebeta, ew2, eb2,
      ow1, ob1, og, obeta, ow2, ob2)


def kernel(grid_vertices, grid_features, point_vertices, point_features,
           eW1, eb1, eg, ebeta, eW2, eb2,
           oW1, ob1, og, obeta, oW2, ob2):
    f32 = jnp.float32
    qv = point_vertices * SCALE
    inv = grid_vertices * SCALE

    qt = jnp.zeros((8, NPP), f32).at[:3, :NPTS].set(qv.T)
    rp = jnp.zeros((NG, 8), f32).at[:, :3].set(inv)
    idx = _knn(qt, rp)

    table = (jnp.zeros((NG, TW), f32)
             .at[:, :128].set(grid_features)
             .at[:, 128:131].set(inv))
    g = _sc_gather(table, idx.reshape(-1))
    g3 = g.reshape(KNN, NPP, TW)

    pf_pad = jnp.pad(point_features, ((0, NPP - NPTS), (0, 0)))
    qv16 = jnp.zeros((NPP, 16), f32).at[:NPTS, :3].set(qv)
    w1cat = (jnp.zeros((TW, 256), f32)
             .at[:128].set(eW1[:128])
             .at[128:131].set(eW1[256:259]))
    w1self = eW1[128:256]
    w1r16 = jnp.zeros((16, 256), f32).at[:3].set(eW1[256:259])

    out = _mlp(g3, pf_pad, qv16, w1cat, w1self, w1r16,
               eb1[None], eg[None], ebeta[None], eW2, eb2[None],
               oW1, ob1[None], og[None], obeta[None], oW2, ob2[None])
    return out[:NPTS]


# speedup vs baseline: 10.8032x; 1.0236x over previous
import functools

import jax
import jax.numpy as jnp
from jax.experimental import pallas as pl
from jax.experimental.pallas import tpu as pltpu
from jax.experimental.pallas import tpu_sc as plsc

NG = 32768
NPTS = 10000
NPP = 10240
KNN = 16
BQ = 256
NBLK = NPP // BQ
RCH = 256
NCH = NG // RCH
TOPC = 3
NCAND = NCH * TOPC
TW = 144
SCALE = 32.0


def _knn_body(qt_ref, rp_ref, out_ref, ck_ref):
    big_i = jnp.int32(2 ** 31 - 1)
    lidx = jax.lax.broadcasted_iota(jnp.int32, (RCH, BQ), 0)

    def chunk(c):
        d = None
        for cc in range(3):
            rc = rp_ref[pl.ds(c * RCH, RCH), cc:cc + 1]
            qc = qt_ref[cc:cc + 1, :]
            t = rc - qc
            d = t * t if d is None else d + t * t
        db = jax.lax.bitcast_convert_type(d, jnp.int32)
        key = jnp.bitwise_or(jnp.bitwise_and(db, jnp.int32(-256)), lidx)
        for j in range(TOPC):
            m = jnp.min(key, axis=0, keepdims=True)
            ck_ref[pl.ds(c * TOPC + j, 1), :] = m
            if j + 1 < TOPC:
                key = jnp.where(key == m, big_i, key)

    for c in range(NCH):
        chunk(c)

    ck = ck_ref[...]
    sio = jax.lax.broadcasted_iota(jnp.int32, (NCAND, BQ), 0)
    for j in range(KNN):
        m = jnp.min(ck, axis=0, keepdims=True)
        aslot = jnp.min(jnp.where(ck == m, sio, big_i),
                        axis=0, keepdims=True)
        gidx = (aslot // TOPC) * RCH + jnp.bitwise_and(m, jnp.int32(255))
        out_ref[pl.ds(j, 1), :] = gidx
        if j + 1 < KNN:
            ck = jnp.where(sio == aslot, big_i, ck)


def _knn(qt, rp):
    return pl.pallas_call(
        _knn_body,
        grid=(NBLK,),
        in_specs=[
            pl.BlockSpec((8, BQ), lambda i: (0, i)),
            pl.BlockSpec((NG, 8), lambda i: (0, 0)),
        ],
        out_specs=pl.BlockSpec((KNN, BQ), lambda i: (0, i)),
        out_shape=jax.ShapeDtypeStruct((KNN, NPP), jnp.int32),
        scratch_shapes=[
            pltpu.VMEM((NCAND, BQ), jnp.int32),
        ],
    )(qt, rp)


def _sc_gather(table, idx2d):
    info = plsc.get_sparse_core_info()
    nw = info.num_cores * info.num_subcores
    n_total, ch = idx2d.shape
    n_iter = n_total // nw
    e = n_total * ch
    mesh = plsc.VectorSubcoreMesh(core_axis_name="c", subcore_axis_name="s")

    @functools.partial(
        pl.kernel, mesh=mesh,
        compiler_params=pltpu.CompilerParams(use_tc_tiling_on_sc=False),
        out_type=jax.ShapeDtypeStruct((e, TW), jnp.float32),
        scratch_types=[
            pltpu.VMEM((n_iter, ch), jnp.int32),
            pltpu.VMEM((ch, TW), jnp.float32),
            pltpu.VMEM((ch, TW), jnp.float32),
            pltpu.SemaphoreType.DMA,
            pltpu.SemaphoreType.DMA,
            pltpu.SemaphoreType.DMA,
            pltpu.SemaphoreType.DMA,
        ],
    )
    def k(table_hbm, idx_hbm, out_hbm, idx_v, buf0, buf1, gs0, gs1, ws0, ws1):
        wid = (jax.lax.axis_index("s") * info.num_cores
               + jax.lax.axis_index("c"))
        base = wid * n_iter
        pltpu.sync_copy(idx_hbm.at[pl.ds(base, n_iter)], idx_v)
        bufs = (buf0, buf1)
        gsem = (gs0, gs1)
        wsem = (ws0, ws1)
        pend = [None, None]
        for i in range(n_iter):
            s = i & 1
            if pend[s] is not None:
                pend[s].wait()
            pltpu.async_copy(table_hbm.at[idx_v.at[i]], bufs[s],
                             gsem[s]).wait()
            pend[s] = pltpu.async_copy(
                bufs[s], out_hbm.at[pl.ds((base + i) * ch, ch)], wsem[s])
        pend[0].wait()
        pend[1].wait()

    return k(table, idx2d)


def _ln_gelu(h, g, beta):
    mu = jnp.mean(h, axis=-1, keepdims=True)
    var = jnp.mean((h - mu) ** 2, axis=-1, keepdims=True)
    h = (h - mu) / jnp.sqrt(var + 1e-5) * g + beta
    return jax.nn.gelu(h)


def _mlp_body(g3_ref, pf_ref, qv_ref,
              w1cat_ref, w1self_ref, w1r_ref, eb1_ref, eg_ref, ebeta_ref,
              ew2_ref, eb2_ref, ow1_ref, ob1_ref, og_ref, obeta_ref,
              ow2_ref, ob2_ref, out_ref):
    mm = functools.partial(jnp.dot, preferred_element_type=jnp.float32)
    pf = pf_ref[...]
    qv = qv_ref[...]
    hself = (mm(pf, w1self_ref[...]) - mm(qv, w1r_ref[...])
             + eb1_ref[...])
    w1cat = w1cat_ref[...]
    ew2 = ew2_ref[...]
    eg = eg_ref[...]
    ebeta = ebeta_ref[...]
    acc = jnp.zeros((BQ, 128), jnp.float32)
    for k in range(KNN):
        gk = g3_ref[k]
        h = mm(gk, w1cat) + hself
        h = _ln_gelu(h, eg, ebeta)
        acc = acc + mm(h, ew2)
    red = acc * (1.0 / KNN) + eb2_ref[...]
    h2 = mm(red, ow1_ref[...]) + ob1_ref[...]
    h2 = _ln_gelu(h2, og_ref[...], obeta_ref[...])
    out_ref[...] = mm(h2, ow2_ref[...]) + ob2_ref[...]


def _mlp(g3, pf, qv16, w1cat, w1self, w1r16, eb1, eg, ebeta, ew2, eb2,
         ow1, ob1, og, obeta, ow2, ob2):
    full = lambda shape: pl.BlockSpec(shape, lambda i: tuple(0 for _ in shape))
    return pl.pallas_call(
        _mlp_body,
        grid=(NBLK,),
        in_specs=[
            pl.BlockSpec((KNN, BQ, TW), lambda i: (0, i, 0)),
            pl.BlockSpec((BQ, 128), lambda i: (i, 0)),
            pl.BlockSpec((BQ, 16), lambda i: (i, 0)),
            full((TW, 256)), full((128, 256)), full((16, 256)),
            full((1, 256)), full((1, 256)), full((1, 256)),
            full((256, 128)), full((1, 128)),
            full((128, 256)), full((1, 256)), full((1, 256)), full((1, 256)),
            full((256, 128)), full((1, 128)),
        ],
        out_specs=pl.BlockSpec((BQ, 128), lambda i: (i, 0)),
        out_shape=jax.ShapeDtypeStruct((NPP, 128), jnp.float32),
    )(g3, pf, qv16, w1cat, w1self, w1r16, eb1, eg, ebeta, ew2, eb2,
      ow1, ob1, og, obeta, ow2, ob2)


def kernel(grid_vertices, grid_features, point_vertices, point_features,
           eW1, eb1, eg, ebeta, eW2, eb2,
           oW1, ob1, og, obeta, oW2, ob2):
    f32 = jnp.float32
    qv = point_vertices * SCALE
    inv = grid_vertices * SCALE

    qt = jnp.zeros((8, NPP), f32).at[:3, :NPTS].set(qv.T)
    rp = jnp.zeros((NG, 8), f32).at[:, :3].set(inv)
    idx = _knn(qt, rp)

    table = (jnp.zeros((NG, TW), f32)
             .at[:, :128].set(grid_features)
             .at[:, 128:131].set(inv))
    g = _sc_gather(table, idx.reshape(-1, 256))
    g3 = g.reshape(KNN, NPP, TW)

    pf_pad = jnp.pad(point_features, ((0, NPP - NPTS), (0, 0)))
    qv16 = jnp.zeros((NPP, 16), f32).at[:NPTS, :3].set(qv)
    w1cat = (jnp.zeros((TW, 256), f32)
             .at[:128].set(eW1[:128])
             .at[128:131].set(eW1[256:259]))
    w1self = eW1[128:256]
    w1r16 = jnp.zeros((16, 256), f32).at[:3].set(eW1[256:259])

    out = _mlp(g3, pf_pad, qv16, w1cat, w1self, w1r16,
               eb1[None], eg[None], ebeta[None], eW2, eb2[None],
               oW1, ob1[None], og[None], obeta[None], oW2, ob2[None])
    return out[:NPTS]
